# baseline (speedup 1.0000x reference)
import jax
import jax.numpy as jnp
from jax.experimental import pallas as pl
from jax.experimental.pallas import tpu as pltpu


def _make_kernel(num_slices, bs):
    def _fused_kernel(*refs):
        x_ref, w_ref = refs[0], refs[1]
        adj_refs = refs[2:2 + num_slices]
        b_ref, out_ref, xw_ref = refs[2 + num_slices:]

        @pl.when(pl.program_id(0) == 0)
        def _():
            xw_ref[...] = jnp.dot(x_ref[...], w_ref[...],
                                  preferred_element_type=jnp.float32)

        for i, a_ref in enumerate(adj_refs):
            acc = jnp.dot(a_ref[...], xw_ref[...],
                          preferred_element_type=jnp.float32)
            out_ref[i * bs:(i + 1) * bs, :] = jnp.maximum(
                acc + b_ref[...], 0.0)

    return _fused_kernel


def kernel(input, adj, W, b):
    x = input.reshape(input.shape[-2], input.shape[-1])
    n, d_in = x.shape
    d_out = W.shape[1]

    bm = min(400, n)
    num_slices = 5 if bm % (5 * 8) == 0 else 1
    bs = bm // num_slices

    def _slice_spec(i):
        return pl.BlockSpec(
            (bs, n), lambda m, i=i: (m * num_slices + i, 0))

    out = pl.pallas_call(
        _make_kernel(num_slices, bs),
        grid=(n // bm,),
        in_specs=[
            pl.BlockSpec((n, d_in), lambda m: (0, 0)),
            pl.BlockSpec((d_in, d_out), lambda m: (0, 0)),
            *[_slice_spec(i) for i in range(num_slices)],
            pl.BlockSpec((1, d_out), lambda m: (0, 0)),
        ],
        out_specs=pl.BlockSpec((bm, d_out), lambda m: (m, 0)),
        out_shape=jax.ShapeDtypeStruct((n, d_out), jnp.float32),
        scratch_shapes=[pltpu.VMEM((n, d_out), jnp.float32)],
    )(x, W, *([adj] * num_slices), b.reshape(1, d_out))

    return out[None]


# speedup vs baseline: 1.0235x; 1.0235x over previous
import jax
import jax.numpy as jnp
from jax.experimental import pallas as pl
from jax.experimental.pallas import tpu as pltpu


def _fused_kernel(x_ref, w_ref, adj0_ref, adj1_ref, b_ref, out_ref, xw_ref):
    @pl.when(pl.program_id(0) == 0)
    def _():
        xw_ref[...] = jnp.dot(x_ref[...], w_ref[...],
                              preferred_element_type=jnp.float32)

    hs = adj0_ref.shape[0]
    acc0 = jnp.dot(adj0_ref[...], xw_ref[...],
                   preferred_element_type=jnp.float32)
    out_ref[:hs, :] = jnp.maximum(acc0 + b_ref[...], 0.0)
    acc1 = jnp.dot(adj1_ref[...], xw_ref[...],
                   preferred_element_type=jnp.float32)
    out_ref[hs:, :] = jnp.maximum(acc1 + b_ref[...], 0.0)


def kernel(input, adj, W, b):
    x = input.reshape(input.shape[-2], input.shape[-1])
    n, d_in = x.shape
    d_out = W.shape[1]

    bm = min(400, n)
    out = pl.pallas_call(
        _fused_kernel,
        grid=(n // bm,),
        in_specs=[
            pl.BlockSpec((n, d_in), lambda m: (0, 0)),
            pl.BlockSpec((d_in, d_out), lambda m: (0, 0)),
            pl.BlockSpec((bm // 2, n), lambda m: (2 * m, 0)),
            pl.BlockSpec((bm // 2, n), lambda m: (2 * m + 1, 0)),
            pl.BlockSpec((1, d_out), lambda m: (0, 0)),
        ],
        out_specs=pl.BlockSpec((bm, d_out), lambda m: (m, 0)),
        out_shape=jax.ShapeDtypeStruct((n, d_out), jnp.float32),
        scratch_shapes=[pltpu.VMEM((n, d_out), jnp.float32)],
    )(x, W, adj, adj, b.reshape(1, d_out))

    return out[None]
